# Initial kernel scaffold; baseline (speedup 1.0000x reference)
import jax
import jax.numpy as jnp
from jax.experimental import pallas as pl


def kernel(x, Q, embeddings):
    raise NotImplementedError("write your pallas kernel here")


# speedup vs baseline: 1.7023x; 1.7023x over previous
import functools

import jax
import jax.numpy as jnp
import numpy as np
from jax.experimental import pallas as pl

B = 16384
D = 256
T = 26
P = 128
K = T * P
COMMIT = 0.25
TEMP = 0.07

ROWS = 256
GRID = B // ROWS


def _main_body(x_ref, q_ref, emb_ref, qst_ref, enc_ref, sq_ref):
    i = pl.program_id(0)
    xt = x_ref[...]
    qv = q_ref[0, 0, :]
    emb = emb_ref[...]

    norm = jnp.sqrt(jnp.sum(xt * xt, axis=1, keepdims=True))
    xn = xt / jnp.maximum(norm, 1e-12)

    xsq = jnp.sum(xn * xn, axis=1, keepdims=True)
    esq = jnp.sum(emb * emb, axis=1)

    s = jax.lax.dot_general(xn, emb, (((1,), (1,)), ((), ())),
                            preferred_element_type=jnp.float32)
    d = xsq + esq[None, :] - 2.0 * s

    oh_t = (qv[:, None] == jax.lax.broadcasted_iota(jnp.int32, (ROWS, T), 1))
    oh_t = oh_t.astype(jnp.float32)
    per = jnp.zeros((ROWS, P), dtype=jnp.float32)
    for t in range(T):
        per = per + d[:, t * P:(t + 1) * P] * oh_t[:, t][:, None]

    li = jnp.argmin(per, axis=1).astype(jnp.int32)
    enc = qv * P + li

    ohk = (enc[:, None] == jax.lax.broadcasted_iota(jnp.int32, (ROWS, K), 1))
    ohk = ohk.astype(jnp.float32)
    qrow = jax.lax.dot_general(ohk, emb, (((1,), (0,)), ((), ())),
                               preferred_element_type=jnp.float32)
    qn = jnp.sqrt(jnp.sum(qrow * qrow, axis=1, keepdims=True))
    qrow = qrow / jnp.maximum(qn, 1e-12)

    qst = xn + (qrow - xn)
    qst_ref[...] = qst
    enc_ref[0, 0, :] = enc

    diff = qrow - xn
    part = jnp.sum(diff * diff).reshape(1, 1)

    @pl.when(i == 0)
    def _():
        sq_ref[...] = jnp.zeros((1, 1), jnp.float32)

    sq_ref[...] += part


def _uniform_body(emb_ref, sel_ref, lab_ref, noteye_ref, valid_ref, out_ref):
    emb = emb_ref[...]
    nrm = jnp.sqrt(jnp.sum(emb * emb, axis=1, keepdims=True))
    nemb = emb / jnp.maximum(nrm, 1e-12)
    se = jax.lax.dot_general(sel_ref[...], nemb, (((1,), (0,)), ((), ())),
                             preferred_element_type=jnp.float32)
    sim = jax.lax.dot_general(se, se, (((1,), (1,)), ((), ())),
                              preferred_element_type=jnp.float32)
    e = jnp.exp(sim / TEMP) * noteye_ref[...]
    sum_exp = jnp.sum(e, axis=1, keepdims=True)
    pos_sum = jnp.sum(e * lab_ref[...], axis=1, keepdims=True)
    valid = valid_ref[...]
    term = jnp.where(valid > 0.0,
                     jnp.log(pos_sum / jnp.maximum(sum_exp, 1e-30)
                             + 1e-45),
                     0.0)
    n_valid = jnp.sum(valid)
    out_ref[...] = (-jnp.sum(term * valid) / n_valid).reshape(1, 1)


@functools.partial(jax.jit, static_argnums=())
def kernel(x, Q, embeddings):
    Q3 = Q.reshape(GRID, 1, ROWS)

    qst, enc3, sqsum = pl.pallas_call(
        _main_body,
        grid=(GRID,),
        in_specs=[
            pl.BlockSpec((ROWS, D), lambda i: (i, 0)),
            pl.BlockSpec((1, 1, ROWS), lambda i: (i, 0, 0)),
            pl.BlockSpec((K, D), lambda i: (0, 0)),
        ],
        out_specs=[
            pl.BlockSpec((ROWS, D), lambda i: (i, 0)),
            pl.BlockSpec((1, 1, ROWS), lambda i: (i, 0, 0)),
            pl.BlockSpec((1, 1), lambda i: (0, 0)),
        ],
        out_shape=[
            jax.ShapeDtypeStruct((B, D), jnp.float32),
            jax.ShapeDtypeStruct((GRID, 1, ROWS), jnp.int32),
            jax.ShapeDtypeStruct((1, 1), jnp.float32),
        ],
    )(x, Q3, embeddings)

    loss = (1.0 + COMMIT) * (sqsum[0, 0] / (B * D))

    sampled_num = int(0.1 * P)
    perm = jax.random.permutation(jax.random.key(42), P)[:sampled_num]
    all_idx = jnp.arange(K).reshape(T, P)
    sampled_indices = all_idx[:, perm].reshape(-1)
    S = T * sampled_num
    SP = 384
    sel = (sampled_indices[:, None] ==
           jnp.arange(K)[None, :]).astype(jnp.float32)
    sel = jnp.pad(sel, ((0, SP - S), (0, 0)))
    labels = sampled_indices // P
    lab = (labels[None, :] == labels[:, None]).astype(jnp.float32)
    lab = jnp.pad(lab, ((0, SP - S), (0, SP - S)))
    noteye = 1.0 - jnp.eye(SP, dtype=jnp.float32)
    colvalid = jnp.pad(jnp.ones((S,), jnp.float32), (0, SP - S))
    noteye = noteye * colvalid[None, :] * colvalid[:, None]
    valid = colvalid[:, None]

    uni = pl.pallas_call(
        _uniform_body,
        out_shape=jax.ShapeDtypeStruct((1, 1), jnp.float32),
    )(embeddings, sel, lab, noteye, valid)

    return (qst, loss, uni[0, 0], enc3.reshape(B))
